# baseline (speedup 1.0000x reference)
import functools

import jax
import jax.numpy as jnp
from jax import lax
from jax.experimental import pallas as pl
from jax.experimental.pallas import tpu as pltpu
from jax.experimental.pallas import tpu_sc as plsc

_NC = 2
_NS = 16
_NW = _NC * _NS
_NB = 4


def _gather_call(batch, hidden_pad, hist_pad):
    mesh = plsc.VectorSubcoreMesh(core_axis_name="c", subcore_axis_name="s")
    per_w = batch // _NW

    @functools.partial(
        pl.kernel,
        mesh=mesh,
        compiler_params=pltpu.CompilerParams(use_tc_tiling_on_sc=False),
        out_type=jax.ShapeDtypeStruct((batch, hist_pad, hidden_pad), jnp.float32),
        scratch_types=[
            pltpu.VMEM((per_w, hist_pad), jnp.int32),
            pltpu.VMEM((_NB, hist_pad, hidden_pad), jnp.float32),
            pltpu.SemaphoreType.DMA,
            pltpu.SemaphoreType.DMA,
            pltpu.SemaphoreType.DMA,
            pltpu.SemaphoreType.DMA,
            pltpu.SemaphoreType.DMA,
            pltpu.SemaphoreType.DMA,
            pltpu.SemaphoreType.DMA,
            pltpu.SemaphoreType.DMA,
        ],
    )
    def run(idx_hbm, tab_hbm, out_hbm, idx_v, banks, g0, g1, g2, g3, s0, s1, s2, s3):
        gsem = (g0, g1, g2, g3)
        ssem = (s0, s1, s2, s3)
        wid = lax.axis_index("s") * _NC + lax.axis_index("c")
        base = wid * per_w
        pltpu.sync_copy(idx_hbm.at[pl.ds(base, per_w)], idx_v)

        def fire_gather(i, p):
            pltpu.async_copy(tab_hbm.at[idx_v.at[i]], banks.at[p], gsem[p])

        def drain_gather(p):
            pltpu.make_async_copy(
                tab_hbm.at[idx_v.at[0]], banks.at[p], gsem[p]
            ).wait()

        def fire_store(i, p):
            pltpu.async_copy(banks.at[p], out_hbm.at[base + i], ssem[p])

        def drain_store(p):
            pltpu.make_async_copy(
                banks.at[p], out_hbm.at[base], ssem[p]
            ).wait()

        def step(i, p, fire_next, drain_prev):
            if drain_prev:
                drain_store((p + 1) % _NB)
            if fire_next:
                fire_gather(i + 1, (p + 1) % _NB)
            drain_gather(p)
            fire_store(i, p)

        fire_gather(0, 0)
        step(0, 0, True, False)
        step(1, 1, True, False)
        step(2, 2, True, False)
        step(3, 3, True, True)

        def body(it, carry):
            i = it * _NB
            step(i, 0, True, True)
            step(i + 1, 1, True, True)
            step(i + 2, 2, True, True)
            step(i + 3, 3, True, True)
            return carry

        lax.fori_loop(1, per_w // _NB - 1, body, 0)
        last = per_w - _NB
        step(last, 0, True, True)
        step(last + 1, 1, True, True)
        step(last + 2, 2, True, True)
        step(last + 3, 3, False, True)
        drain_store(1)
        drain_store(2)
        drain_store(3)

    return run


def kernel(indices, embeddings):
    batch, hist = indices.shape
    num_rows, hidden = embeddings.shape
    hidden_pad = 128
    hist_pad = (hist + 7) // 8 * 8
    tab_pad = jnp.pad(embeddings, ((0, 0), (0, hidden_pad - hidden)))
    idx = jnp.pad(indices.astype(jnp.int32), ((0, 0), (0, hist_pad - hist)))
    out = _gather_call(batch, hidden_pad, hist_pad)(idx, tab_pad)
    return out[:, :hist, :hidden]


# speedup vs baseline: 1.0007x; 1.0007x over previous
import functools

import jax
import jax.numpy as jnp
from jax import lax
from jax.experimental import pallas as pl
from jax.experimental.pallas import tpu as pltpu
from jax.experimental.pallas import tpu_sc as plsc

_NC = 2
_NS = 16
_NW = _NC * _NS
_NB = 6
_AHEAD = 3


def _gather_call(batch, hidden_pad, hist_pad):
    mesh = plsc.VectorSubcoreMesh(core_axis_name="c", subcore_axis_name="s")
    per_w = batch // _NW
    steps = per_w // 2
    rows = 2 * hist_pad

    @functools.partial(
        pl.kernel,
        mesh=mesh,
        compiler_params=pltpu.CompilerParams(use_tc_tiling_on_sc=False),
        out_type=jax.ShapeDtypeStruct((batch * hist_pad, hidden_pad), jnp.float32),
        scratch_types=[
            pltpu.VMEM((per_w * hist_pad,), jnp.int32),
            pltpu.VMEM((_NB, rows, hidden_pad), jnp.float32),
            pltpu.SemaphoreType.DMA,
            pltpu.SemaphoreType.DMA,
            pltpu.SemaphoreType.DMA,
            pltpu.SemaphoreType.DMA,
            pltpu.SemaphoreType.DMA,
            pltpu.SemaphoreType.DMA,
            pltpu.SemaphoreType.DMA,
            pltpu.SemaphoreType.DMA,
            pltpu.SemaphoreType.DMA,
            pltpu.SemaphoreType.DMA,
            pltpu.SemaphoreType.DMA,
            pltpu.SemaphoreType.DMA,
        ],
    )
    def run(idx_hbm, tab_hbm, out_hbm, idx_v, banks, *sems):
        gsem = sems[:_NB]
        ssem = sems[_NB:]
        wid = lax.axis_index("s") * _NC + lax.axis_index("c")
        base = wid * per_w * hist_pad
        pltpu.sync_copy(idx_hbm.at[pl.ds(base, per_w * hist_pad)], idx_v)

        def fire_gather(k, p):
            pltpu.async_copy(
                tab_hbm.at[idx_v.at[pl.ds(k * rows, rows)]], banks.at[p], gsem[p]
            )

        def drain_gather(p):
            pltpu.make_async_copy(
                tab_hbm.at[idx_v.at[pl.ds(0, rows)]], banks.at[p], gsem[p]
            ).wait()

        def fire_store(k, p):
            pltpu.async_copy(
                banks.at[p], out_hbm.at[pl.ds(base + k * rows, rows)], ssem[p]
            )

        def drain_store(p):
            pltpu.make_async_copy(
                banks.at[p], out_hbm.at[pl.ds(base, rows)], ssem[p]
            ).wait()

        def step(k, p, fire_next, drain_prev):
            if drain_prev:
                drain_store((p + _AHEAD) % _NB)
            if fire_next:
                fire_gather(k + _AHEAD, (p + _AHEAD) % _NB)
            drain_gather(p)
            fire_store(k, p)

        for p in range(_AHEAD):
            fire_gather(p, p)
        step(0, 0, True, False)
        step(1, 1, True, False)
        step(2, 2, True, False)

        def body(it, carry):
            k = it * _NB + _AHEAD
            for off in range(_NB):
                step(k + off, (off + _AHEAD) % _NB, True, True)
            return carry

        n_body = (steps - _AHEAD * 2 - 1) // _NB
        lax.fori_loop(0, n_body, body, 0)
        for k in range(_AHEAD + n_body * _NB, steps):
            step(k, k % _NB, k + _AHEAD < steps, True)
        for k in range(steps - _AHEAD, steps):
            drain_store(k % _NB)

    return run


def kernel(indices, embeddings):
    batch, hist = indices.shape
    num_rows, hidden = embeddings.shape
    hidden_pad = 128
    hist_pad = (hist + 7) // 8 * 8
    tab_pad = jnp.pad(embeddings, ((0, 0), (0, hidden_pad - hidden)))
    idx = jnp.pad(indices.astype(jnp.int32), ((0, 0), (0, hist_pad - hist)))
    out = _gather_call(batch, hidden_pad, hist_pad)(idx.reshape(-1), tab_pad)
    out = out.reshape(batch, hist_pad, hidden_pad)
    return out[:, :hist, :hidden]


# speedup vs baseline: 1.7255x; 1.7242x over previous
import functools

import jax
import jax.numpy as jnp
from jax import lax
from jax.experimental import pallas as pl
from jax.experimental.pallas import tpu as pltpu
from jax.experimental.pallas import tpu_sc as plsc

_NC = 2
_NS = 16
_NW = _NC * _NS
_NB = 6
_AHEAD = 3


def _gather_call(batch, hidden_pad, hist_pad):
    mesh = plsc.VectorSubcoreMesh(core_axis_name="c", subcore_axis_name="s")
    per_w = batch // _NW
    steps = per_w // 2
    rows = 2 * hist_pad

    @functools.partial(
        pl.kernel,
        mesh=mesh,
        compiler_params=pltpu.CompilerParams(use_tc_tiling_on_sc=False),
        out_type=jax.ShapeDtypeStruct((batch * hist_pad, hidden_pad), jnp.float32),
        scratch_types=[
            pltpu.VMEM((per_w * hist_pad,), jnp.int32),
            pltpu.VMEM((_NB, rows, 64), jnp.float32),
            pltpu.SemaphoreType.DMA,
            pltpu.SemaphoreType.DMA,
            pltpu.SemaphoreType.DMA,
            pltpu.SemaphoreType.DMA,
            pltpu.SemaphoreType.DMA,
            pltpu.SemaphoreType.DMA,
            pltpu.SemaphoreType.DMA,
            pltpu.SemaphoreType.DMA,
            pltpu.SemaphoreType.DMA,
            pltpu.SemaphoreType.DMA,
            pltpu.SemaphoreType.DMA,
            pltpu.SemaphoreType.DMA,
        ],
    )
    def run(idx_hbm, tab_hbm, out_hbm, idx_v, banks, *sems):
        gsem = sems[:_NB]
        ssem = sems[_NB:]
        wid = lax.axis_index("s") * _NC + lax.axis_index("c")
        base = wid * per_w * hist_pad
        pltpu.sync_copy(idx_hbm.at[pl.ds(base, per_w * hist_pad)], idx_v)

        def fire_gather(k, p):
            pltpu.async_copy(
                tab_hbm.at[idx_v.at[pl.ds(k * rows, rows)]], banks.at[p], gsem[p]
            )

        def drain_gather(p):
            pltpu.make_async_copy(
                tab_hbm.at[idx_v.at[pl.ds(0, rows)]], banks.at[p], gsem[p]
            ).wait()

        def fire_store(k, p):
            pltpu.async_copy(
                banks.at[p],
                out_hbm.at[pl.ds(base + k * rows, rows), pl.ds(0, 64)],
                ssem[p],
            )

        def drain_store(p):
            pltpu.make_async_copy(
                banks.at[p], out_hbm.at[pl.ds(base, rows), pl.ds(0, 64)], ssem[p]
            ).wait()

        def step(k, p, fire_next, drain_prev):
            if drain_prev:
                drain_store((p + _AHEAD) % _NB)
            if fire_next:
                fire_gather(k + _AHEAD, (p + _AHEAD) % _NB)
            drain_gather(p)
            fire_store(k, p)

        for p in range(_AHEAD):
            fire_gather(p, p)
        step(0, 0, True, False)
        step(1, 1, True, False)
        step(2, 2, True, False)

        def body(it, carry):
            k = it * _NB + _AHEAD
            for off in range(_NB):
                step(k + off, (off + _AHEAD) % _NB, True, True)
            return carry

        n_body = (steps - _AHEAD * 2 - 1) // _NB
        lax.fori_loop(0, n_body, body, 0)
        for k in range(_AHEAD + n_body * _NB, steps):
            step(k, k % _NB, k + _AHEAD < steps, True)
        for k in range(steps - _AHEAD, steps):
            drain_store(k % _NB)

    return run


def kernel(indices, embeddings):
    batch, hist = indices.shape
    num_rows, hidden = embeddings.shape
    hidden_pad = 128
    hist_pad = (hist + 7) // 8 * 8
    tab_pad = embeddings
    idx = jnp.pad(indices.astype(jnp.int32), ((0, 0), (0, hist_pad - hist)))
    out = _gather_call(batch, hidden_pad, hist_pad)(idx.reshape(-1), tab_pad)
    out = out.reshape(batch, hist_pad, hidden_pad)
    return out[:, :hist, :hidden]


# speedup vs baseline: 2.9487x; 1.7089x over previous
import functools

import jax
import jax.numpy as jnp
from jax import lax
from jax.experimental import pallas as pl
from jax.experimental.pallas import tpu as pltpu
from jax.experimental.pallas import tpu_sc as plsc

_NC = 2
_NS = 16
_NW = _NC * _NS
_BB = 128
_NB = 3
_L = 16


def _gather_call(batch, hist, hidden):
    mesh = plsc.VectorSubcoreMesh(core_axis_name="c", subcore_axis_name="s")
    per_w = batch // _NW
    nblk_b = per_w // _BB
    steps = hist * nblk_b
    dh = hidden // 8
    blk_elems = hidden * _BB

    @functools.partial(
        pl.kernel,
        mesh=mesh,
        compiler_params=pltpu.CompilerParams(
            use_tc_tiling_on_sc=False, needs_layout_passes=False
        ),
        out_type=jax.ShapeDtypeStruct((hist, dh, batch // _BB, 8, _BB), jnp.float32),
        scratch_types=[
            pltpu.VMEM((per_w * hist,), jnp.int32),
            pltpu.VMEM((_NB, _BB), jnp.int32),
            pltpu.VMEM((_NB, _BB, hidden), jnp.float32),
            pltpu.VMEM((_NB, dh, 8, _BB), jnp.float32),
            pltpu.SemaphoreType.DMA,
            pltpu.SemaphoreType.DMA,
            pltpu.SemaphoreType.DMA,
            pltpu.SemaphoreType.DMA,
            pltpu.SemaphoreType.DMA,
            pltpu.SemaphoreType.DMA,
        ],
    )
    def run(idx_hbm, tab_hbm, out_hbm, idx_v, idxl, rows, trans, *sems):
        gsem = sems[:_NB]
        ssem = sems[_NB:]
        wid = lax.axis_index("s") * _NC + lax.axis_index("c")
        b0w = wid * per_w
        pltpu.sync_copy(idx_hbm.at[pl.ds(b0w * hist, per_w * hist)], idx_v)

        iota = lax.iota(jnp.int32, _L)
        iota_h = iota * hist
        iota_r = iota

        def build_idx(k, p):
            t = k // nblk_b
            bb = (k % nblk_b) * _BB
            for v in range(_BB // _L):
                pos = iota_h + ((bb + v * _L) * hist + t)
                idxl[p, pl.ds(v * _L, _L)] = plsc.load_gather(idx_v, [pos])

        def fire_gather(k, p):
            build_idx(k, p)
            pltpu.async_copy(tab_hbm.at[idxl.at[p]], rows.at[p], gsem[p])

        def drain_gather(p):
            pltpu.make_async_copy(
                tab_hbm.at[idxl.at[0]], rows.at[p], gsem[p]
            ).wait()

        def transpose(p):
            def dloop(d, carry):
                dlo = d % 8
                dhi = d // 8
                dsplat = jnp.full((_L,), d, jnp.int32)
                for v in range(_BB // _L):
                    g = plsc.load_gather(
                        rows, [jnp.full((_L,), p, jnp.int32), iota_r + v * _L, dsplat]
                    )
                    trans[p, dhi, dlo, pl.ds(v * _L, _L)] = g
                return carry

            lax.fori_loop(0, hidden, dloop, 0)

        def fire_store(k, p):
            t = k // nblk_b
            col = wid * nblk_b + k % nblk_b
            pltpu.async_copy(
                trans.at[p], out_hbm.at[t, :, col, :, :], ssem[p]
            )

        def drain_store(p):
            pltpu.make_async_copy(
                trans.at[p], out_hbm.at[0, :, 0, :, :], ssem[p]
            ).wait()

        def step(k, p, fire_next, drain_prev):
            if drain_prev:
                drain_store(p)
            if fire_next:
                fire_gather(k + 2, (p + 2) % _NB)
            drain_gather(p)
            transpose(p)
            fire_store(k, p)

        fire_gather(0, 0)
        fire_gather(1, 1)
        step(0, 0, True, False)
        step(1, 1, True, False)
        step(2, 2, True, False)

        def body(it, carry):
            k = it * _NB
            step(k, 0, True, True)
            step(k + 1, 1, True, True)
            step(k + 2, 2, True, True)
            return carry

        n_body = (steps - 3 - 2) // _NB
        lax.fori_loop(1, n_body + 1, body, 0)
        for k in range(3 + n_body * _NB, steps):
            step(k, k % _NB, k + 2 < steps, True)
        for p in range(_NB):
            drain_store(p)

    return run


def kernel(indices, embeddings):
    batch, hist = indices.shape
    num_rows, hidden = embeddings.shape
    idx = indices.astype(jnp.int32).reshape(-1)
    out = _gather_call(batch, hist, hidden)(idx, embeddings)
    out = out.transpose(2, 4, 0, 1, 3)
    return out.reshape(batch, hist, hidden)
